# baseline (speedup 1.0000x reference)
import jax
import jax.numpy as jnp
from jax import lax
from jax.experimental import pallas as pl
from jax.experimental.pallas import tpu as pltpu
from jax.experimental.pallas import tpu_sc as plsc

SEQ_LEN = 200
EMBED_DIM = 64
NC = 2
NS = 16
NW = NC * NS
NPHASE = 2

H0, H1 = 104, SEQ_LEN - 104
WIN0 = tuple(range(0, 96, 16)) + (88,)
WIN1 = tuple(range(H0, SEQ_LEN - 16 + 1, 16))


def _body(x_hbm, pos_hbm, tok_hbm, out_hbm,
          idx_v, pos_v, gb0, gb1, gb2, gb3, in0, in1, in2, in3,
          out0, out1, g0, g1, g2, g3, s0, s1):
    gbs, ins = [gb0, gb1, gb2, gb3], [in0, in1, in2, in3]
    outs, gsems, ssems = [out0, out1], [g0, g1, g2, g3], [s0, s1]
    los, lens = (0, H0), (H0, H1)
    rows = x_hbm.shape[0] // NW
    prows = rows // NPHASE
    wid = lax.axis_index("s") * NC + lax.axis_index("c")
    row_base = wid * rows

    pltpu.sync_copy(pos_hbm, pos_v)

    def fill_gidx(t, b):
        p = b % 2
        for o in (WIN0, WIN1)[p]:
            gbs[b][pl.ds(o - los[p], 16)] = lax.shift_right_logical(
                idx_v[t, pl.ds(o, 16)], 1)

    def gather_half(b):
        pltpu.async_copy(tok_hbm.at[gbs[b].at[pl.ds(0, lens[b % 2])]],
                         ins[b], gsems[b])

    def wait_half(b):
        pltpu.make_async_copy(tok_hbm.at[gbs[b].at[pl.ds(0, lens[b % 2])]],
                              ins[b], gsems[b]).wait()

    def add_window(t, b, o, lanes):
        p = b % 2
        toks = idx_v[t, pl.ds(o, 16)]
        for l in lanes:
            r = o + l
            rb = r - los[p]
            odd = (toks[l] & 1) != 0
            for j in range(EMBED_DIM // 16):
                sl = pl.ds(16 * j, 16)
                lo_v = ins[b][rb, sl]
                hi_v = ins[b][rb, pl.ds(EMBED_DIM + 16 * j, 16)]
                outs[p][rb, sl] = jnp.where(odd, hi_v, lo_v) + pos_v[r, sl]

    def phase_body(phase, pcarry):
        pltpu.sync_copy(
            x_hbm.at[pl.ds(row_base + phase * prows, prows)], idx_v)

        for b in range(4):
            fill_gidx(b // 2, b)
            gather_half(b)

        nh = 2 * prows

        def outer(u, carry):
            for k in range(4):
                b, p = k, k % 2
                h = 4 * u + k
                t = 2 * u + (k // 2)
                wait_half(b)

                def wait_store():
                    pltpu.make_async_copy(
                        outs[p], out_hbm.at[0, pl.ds(0, lens[p])],
                        ssems[p]).wait()
                if k >= 2:
                    wait_store()
                else:
                    pl.when(u >= 1)(wait_store)

                if p == 0:
                    @plsc.parallel_loop(0, len(WIN0) - 1)
                    def _half0(g):
                        add_window(t, b, pl.multiple_of(g * 16, 16),
                                   range(16))
                    add_window(t, b, WIN0[-1], range(8, 16))
                else:
                    for o in WIN1:
                        add_window(t, b, o, range(16))

                pltpu.async_copy(
                    outs[p],
                    out_hbm.at[row_base + phase * prows + t,
                               pl.ds(los[p], lens[p])],
                    ssems[p])

                @pl.when(h + 4 < nh)
                def _next_gather():
                    fill_gidx(t + 2, b)
                    gather_half(b)
            return carry

        lax.fori_loop(0, nh // 4, outer, 0)

        for p in range(2):
            pltpu.make_async_copy(outs[p], out_hbm.at[0, pl.ds(0, lens[p])],
                                  ssems[p]).wait()
        return pcarry

    lax.fori_loop(0, NPHASE, phase_body, 0)


def _make_kernel(batch):
    mesh = plsc.VectorSubcoreMesh(core_axis_name="c", subcore_axis_name="s")
    rows = batch // NW
    return pl.kernel(
        _body,
        out_type=jax.ShapeDtypeStruct((batch, SEQ_LEN, EMBED_DIM), jnp.float32),
        mesh=mesh,
        scratch_types=[
            pltpu.VMEM((rows // NPHASE, SEQ_LEN), jnp.int32),
            pltpu.VMEM((SEQ_LEN, EMBED_DIM), jnp.float32),
            pltpu.VMEM((112,), jnp.int32),
            pltpu.VMEM((H1,), jnp.int32),
            pltpu.VMEM((112,), jnp.int32),
            pltpu.VMEM((H1,), jnp.int32),
            pltpu.VMEM((H0, 2 * EMBED_DIM), jnp.float32),
            pltpu.VMEM((H1, 2 * EMBED_DIM), jnp.float32),
            pltpu.VMEM((H0, 2 * EMBED_DIM), jnp.float32),
            pltpu.VMEM((H1, 2 * EMBED_DIM), jnp.float32),
            pltpu.VMEM((H0, EMBED_DIM), jnp.float32),
            pltpu.VMEM((H1, EMBED_DIM), jnp.float32),
            pltpu.SemaphoreType.DMA,
            pltpu.SemaphoreType.DMA,
            pltpu.SemaphoreType.DMA,
            pltpu.SemaphoreType.DMA,
            pltpu.SemaphoreType.DMA,
            pltpu.SemaphoreType.DMA,
        ],
        compiler_params=pltpu.CompilerParams(use_tc_tiling_on_sc=True),
    )


def kernel(x, token_table, pos_table):
    batch, seq = x.shape
    assert seq == SEQ_LEN
    vocab, dim = token_table.shape
    assert dim == EMBED_DIM
    tok_pairs = token_table.reshape(vocab // 2, 2 * EMBED_DIM)
    run = _make_kernel(batch)
    return run(x.astype(jnp.int32), pos_table, tok_pairs)


# speedup vs baseline: 1.3054x; 1.3054x over previous
import jax
import jax.numpy as jnp
from jax import lax
from jax.experimental import pallas as pl
from jax.experimental.pallas import tpu as pltpu
from jax.experimental.pallas import tpu_sc as plsc

SEQ_LEN = 200
EMBED_DIM = 64
NC = 2
NS = 16
NW = NC * NS


def _body(x_hbm, pos_hbm, tok_hbm, out_hbm,
          idx_v, pos_v, in0, in1, out0, out1, g0, g1, s0, s1):
    ins, outs, gsems, ssems = [in0, in1], [out0, out1], [g0, g1], [s0, s1]
    rows = x_hbm.shape[0] // NW
    wid = lax.axis_index("s") * NC + lax.axis_index("c")
    row_base = wid * rows

    pltpu.sync_copy(x_hbm.at[pl.ds(row_base, rows)], idx_v)
    pltpu.sync_copy(pos_hbm, pos_v)

    SPLITS = ((0, 128), (128, SEQ_LEN - 128))

    def gather_row(t, p):
        for lo, n in SPLITS:
            pltpu.async_copy(
                tok_hbm.at[idx_v.at[t, pl.ds(lo, n)]],
                ins[p].at[pl.ds(lo, n)], gsems[p])

    def wait_row(t, p):
        for lo, n in SPLITS:
            pltpu.make_async_copy(
                tok_hbm.at[idx_v.at[t, pl.ds(lo, n)]],
                ins[p].at[pl.ds(lo, n)], gsems[p]).wait()

    for p in range(2):
        gather_row(p, p)

    def outer(u, carry):
        for p in range(2):
            t = 2 * u + p
            wait_row(t, p)

            @pl.when(u >= 1)
            def _wait_store():
                pltpu.make_async_copy(outs[p], out_hbm.at[0], ssems[p]).wait()

            @plsc.parallel_loop(0, SEQ_LEN, unroll=4)
            def row_add(r):
                for j in range(EMBED_DIM // 16):
                    sl = pl.ds(16 * j, 16)
                    outs[p][r, sl] = ins[p][r, sl] + pos_v[r, sl]

            pltpu.async_copy(outs[p], out_hbm.at[row_base + t], ssems[p])

            @pl.when(t + 2 < rows)
            def _next_gather():
                gather_row(t + 2, p)
        return carry

    lax.fori_loop(0, rows // 2, outer, 0)

    for p in range(2):
        pltpu.make_async_copy(outs[p], out_hbm.at[0], ssems[p]).wait()


def _make_kernel(batch):
    mesh = plsc.VectorSubcoreMesh(core_axis_name="c", subcore_axis_name="s")
    rows = batch // NW
    return pl.kernel(
        _body,
        out_type=jax.ShapeDtypeStruct((batch, SEQ_LEN, EMBED_DIM), jnp.float32),
        mesh=mesh,
        scratch_types=[
            pltpu.VMEM((rows, SEQ_LEN), jnp.int32),
            pltpu.VMEM((SEQ_LEN, EMBED_DIM), jnp.float32),
            pltpu.VMEM((SEQ_LEN, EMBED_DIM), jnp.float32),
            pltpu.VMEM((SEQ_LEN, EMBED_DIM), jnp.float32),
            pltpu.VMEM((SEQ_LEN, EMBED_DIM), jnp.float32),
            pltpu.VMEM((SEQ_LEN, EMBED_DIM), jnp.float32),
            pltpu.SemaphoreType.DMA,
            pltpu.SemaphoreType.DMA,
            pltpu.SemaphoreType.DMA,
            pltpu.SemaphoreType.DMA,
        ],
        compiler_params=pltpu.CompilerParams(use_tc_tiling_on_sc=False),
    )


def kernel(x, token_table, pos_table):
    batch, seq = x.shape
    assert seq == SEQ_LEN
    run = _make_kernel(batch)
    return run(x.astype(jnp.int32), pos_table, token_table)


# speedup vs baseline: 1.3064x; 1.0008x over previous
import jax
import jax.numpy as jnp
from jax import lax
from jax.experimental import pallas as pl
from jax.experimental.pallas import tpu as pltpu
from jax.experimental.pallas import tpu_sc as plsc

SEQ_LEN = 200
EMBED_DIM = 64
NC = 2
NS = 16
NW = NC * NS


def _body(x_hbm, pos_hbm, tok_hbm, out_hbm,
          idx_v, pos_v, in0, in1, out0, out1, g0, g1, s0, s1):
    ins, outs, gsems, ssems = [in0, in1], [out0, out1], [g0, g1], [s0, s1]
    rows = x_hbm.shape[0] // NW
    wid = lax.axis_index("s") * NC + lax.axis_index("c")
    row_base = wid * rows

    pltpu.sync_copy(x_hbm.at[pl.ds(row_base, rows)], idx_v)
    pltpu.sync_copy(pos_hbm, pos_v)

    SPLITS = ((0, 128), (128, SEQ_LEN - 128))

    def gather_row(t, p):
        for lo, n in SPLITS:
            pltpu.async_copy(
                tok_hbm.at[idx_v.at[t, pl.ds(lo, n)]],
                ins[p].at[pl.ds(lo, n)], gsems[p])

    def wait_row(t, p):
        for lo, n in SPLITS:
            pltpu.make_async_copy(
                tok_hbm.at[idx_v.at[t, pl.ds(lo, n)]],
                ins[p].at[pl.ds(lo, n)], gsems[p]).wait()

    for p in range(2):
        gather_row(p, p)

    def outer(u, carry):
        for p in range(2):
            t = 2 * u + p
            wait_row(t, p)

            @pl.when(u >= 1)
            def _wait_store():
                pltpu.make_async_copy(outs[p], out_hbm.at[0], ssems[p]).wait()

            @plsc.parallel_loop(0, SEQ_LEN, unroll=8)
            def row_add(r):
                for j in range(EMBED_DIM // 16):
                    sl = pl.ds(16 * j, 16)
                    outs[p][r, sl] = ins[p][r, sl] + pos_v[r, sl]

            pltpu.async_copy(outs[p], out_hbm.at[row_base + t], ssems[p])

            @pl.when(t + 2 < rows)
            def _next_gather():
                gather_row(t + 2, p)
        return carry

    lax.fori_loop(0, rows // 2, outer, 0)

    for p in range(2):
        pltpu.make_async_copy(outs[p], out_hbm.at[0], ssems[p]).wait()


def _make_kernel(batch):
    mesh = plsc.VectorSubcoreMesh(core_axis_name="c", subcore_axis_name="s")
    rows = batch // NW
    return pl.kernel(
        _body,
        out_type=jax.ShapeDtypeStruct((batch, SEQ_LEN, EMBED_DIM), jnp.float32),
        mesh=mesh,
        scratch_types=[
            pltpu.VMEM((rows, SEQ_LEN), jnp.int32),
            pltpu.VMEM((SEQ_LEN, EMBED_DIM), jnp.float32),
            pltpu.VMEM((SEQ_LEN, EMBED_DIM), jnp.float32),
            pltpu.VMEM((SEQ_LEN, EMBED_DIM), jnp.float32),
            pltpu.VMEM((SEQ_LEN, EMBED_DIM), jnp.float32),
            pltpu.VMEM((SEQ_LEN, EMBED_DIM), jnp.float32),
            pltpu.SemaphoreType.DMA,
            pltpu.SemaphoreType.DMA,
            pltpu.SemaphoreType.DMA,
            pltpu.SemaphoreType.DMA,
        ],
        compiler_params=pltpu.CompilerParams(use_tc_tiling_on_sc=False),
    )


def kernel(x, token_table, pos_table):
    batch, seq = x.shape
    assert seq == SEQ_LEN
    run = _make_kernel(batch)
    return run(x.astype(jnp.int32), pos_table, token_table)
